# baseline (device time: 29602 ns/iter reference)
import jax
import jax.numpy as jnp
from jax import lax
from jax.experimental import pallas as pl
from jax.experimental.pallas import tpu as pltpu

N_DEV = 4
B_LOC = 2
SQ = 128
SKV = 128
HB = 64
HQ = 16
H_LOC = 4
DH = 64
D_MODEL = 512
HD_LOC = H_LOC * DH

BF16 = jnp.bfloat16
F32 = jnp.float32


def kernel(x, Wq, K_ext, V_ext, Wo):
    def body(x_ref, wq_ref, k_hbm, v_hbm, wo_ref, out_ref,
             wq_all, wo_all, k_vmem, v_vmem,
             wq_ssem, wq_rsem, wo_ssem, wo_rsem, kv_sem):
        my_pos = lax.axis_index("i")
        left = lax.rem(my_pos + N_DEV - 1, N_DEV)
        right = lax.rem(my_pos + 1, N_DEV)
        jm1 = left
        jp1 = right
        jm2 = lax.rem(my_pos + 2, N_DEV)

        kv_copies = []
        for b in range(B_LOC):
            bg = my_pos * B_LOC + b
            for h in range(HQ):
                for src, dst in ((k_hbm, k_vmem), (v_hbm, v_vmem)):
                    cp = pltpu.make_async_copy(
                        src.at[bg, :, h, :], dst.at[b * HQ + h], kv_sem)
                    cp.start()
                    kv_copies.append(cp)

        wq_all[pl.ds(my_pos, 1)] = wq_ref[...].astype(BF16)[None]
        wo_all[pl.ds(my_pos, 1)] = wo_ref[...].astype(BF16)[None]

        barrier_sem = pltpu.get_barrier_semaphore()
        for nbr in (left, right):
            pl.semaphore_signal(
                barrier_sem, inc=1,
                device_id=(nbr,), device_id_type=pl.DeviceIdType.MESH,
            )
        pl.semaphore_wait(barrier_sem, 2)

        def copy(buf, slot_idx, ssem, rsem, slot, dev):
            return pltpu.make_async_remote_copy(
                src_ref=buf.at[slot_idx],
                dst_ref=buf.at[slot_idx],
                send_sem=ssem.at[slot],
                recv_sem=rsem.at[slot],
                device_id=(dev,),
                device_id_type=pl.DeviceIdType.MESH,
            )

        q_r0 = copy(wq_all, my_pos, wq_ssem, wq_rsem, 0, right)
        o_l0 = copy(wo_all, my_pos, wo_ssem, wo_rsem, 1, left)
        o_r0 = copy(wo_all, my_pos, wo_ssem, wo_rsem, 0, right)
        q_l0 = copy(wq_all, my_pos, wq_ssem, wq_rsem, 1, left)
        q_r0.start()
        o_l0.start()
        o_r0.start()
        q_l0.start()

        q_recv0 = copy(wq_all, jm1, wq_ssem, wq_rsem, 0, left)
        q_recv1 = copy(wq_all, jp1, wq_ssem, wq_rsem, 1, right)
        q_recv2 = copy(wq_all, jm2, wq_ssem, wq_rsem, 2, left)
        o_recv0 = copy(wo_all, jm1, wo_ssem, wo_rsem, 0, left)
        o_recv1 = copy(wo_all, jp1, wo_ssem, wo_rsem, 1, right)
        o_recv2 = copy(wo_all, jm2, wo_ssem, wo_rsem, 2, right)

        for cp in kv_copies:
            cp.wait()

        xs = [(x_ref[b] * 0.125).astype(BF16) for b in range(B_LOC)]

        def block_contrib(j, accs):
            wq_j = wq_all[pl.ds(j, 1)].reshape(D_MODEL, HD_LOC)
            wo_j = wo_all[pl.ds(j, 1)].reshape(HD_LOC, D_MODEL)
            out = []
            for b in range(B_LOC):
                q_blk = lax.dot_general(
                    xs[b], wq_j, (((1,), (0,)), ((), ())),
                    preferred_element_type=F32,
                ).astype(BF16)
                ctx_t, ctx_b = [], []
                for r in range(H_LOC):
                    h_idx = b * HQ + j * H_LOC + r
                    k = k_vmem[pl.ds(h_idx, 1)].reshape(SKV, DH).astype(BF16)
                    v = v_vmem[pl.ds(h_idx, 1)].reshape(SKV, DH).astype(BF16)
                    qt = q_blk[:HB, r * DH:(r + 1) * DH]
                    qb = q_blk[HB:, r * DH:(r + 1) * DH]
                    st = lax.dot_general(
                        qt, k[:HB], (((1,), (1,)), ((), ())),
                        preferred_element_type=F32)
                    sb = lax.dot_general(
                        qb, k, (((1,), (1,)), ((), ())),
                        preferred_element_type=F32)
                    et = jnp.exp(st)
                    eb = jnp.exp(sb)
                    rt = 1.0 / jnp.sum(et, axis=-1, keepdims=True)
                    rb = 1.0 / jnp.sum(eb, axis=-1, keepdims=True)
                    ct = lax.dot_general(
                        et.astype(BF16), v[:HB], (((1,), (0,)), ((), ())),
                        preferred_element_type=F32)
                    cb = lax.dot_general(
                        eb.astype(BF16), v, (((1,), (0,)), ((), ())),
                        preferred_element_type=F32)
                    ctx_t.append((ct * rt).astype(BF16))
                    ctx_b.append((cb * rb).astype(BF16))
                cat_t = jnp.concatenate(ctx_t, axis=1)
                cat_b = jnp.concatenate(ctx_b, axis=1)
                at, ab = accs[b]
                out.append((
                    at + lax.dot_general(
                        cat_t, wo_j, (((1,), (0,)), ((), ())),
                        preferred_element_type=F32),
                    ab + lax.dot_general(
                        cat_b, wo_j, (((1,), (0,)), ((), ())),
                        preferred_element_type=F32),
                ))
            return out

        accs = [(jnp.zeros((HB, D_MODEL), F32),
                 jnp.zeros((HB, D_MODEL), F32)) for _ in range(B_LOC)]

        accs = block_contrib(my_pos, accs)

        q_recv0.wait_recv()
        q_f = copy(wq_all, jm1, wq_ssem, wq_rsem, 2, right)
        q_f.start()
        o_recv1.wait_recv()
        o_f = copy(wo_all, jp1, wo_ssem, wo_rsem, 2, left)
        o_f.start()

        o_recv0.wait_recv()
        accs = block_contrib(jm1, accs)

        q_recv1.wait_recv()
        accs = block_contrib(jp1, accs)

        q_recv2.wait_recv()
        o_recv2.wait_recv()
        accs = block_contrib(jm2, accs)

        for b in range(B_LOC):
            out_ref[b] = jnp.concatenate([accs[b][0], accs[b][1]], axis=0)

        for d in (q_r0, q_l0, o_r0, o_l0, q_f, o_f):
            d.wait_send()

    return pl.pallas_call(
        body,
        out_shape=jax.ShapeDtypeStruct((B_LOC, SQ, D_MODEL), F32),
        in_specs=[
            pl.BlockSpec(memory_space=pltpu.VMEM),
            pl.BlockSpec(memory_space=pltpu.VMEM),
            pl.BlockSpec(memory_space=pl.ANY),
            pl.BlockSpec(memory_space=pl.ANY),
            pl.BlockSpec(memory_space=pltpu.VMEM),
        ],
        out_specs=pl.BlockSpec(memory_space=pltpu.VMEM),
        scratch_shapes=[
            pltpu.VMEM((N_DEV, D_MODEL, HD_LOC), BF16),
            pltpu.VMEM((N_DEV, HD_LOC, D_MODEL), BF16),
            pltpu.VMEM((B_LOC * HQ, SKV, DH), F32),
            pltpu.VMEM((B_LOC * HQ, SKV, DH), F32),
            pltpu.SemaphoreType.DMA((3,)),
            pltpu.SemaphoreType.DMA((3,)),
            pltpu.SemaphoreType.DMA((3,)),
            pltpu.SemaphoreType.DMA((3,)),
            pltpu.SemaphoreType.DMA,
        ],
        compiler_params=pltpu.CompilerParams(collective_id=0),
    )(x, Wq, K_ext, V_ext, Wo)


# device time: 21117 ns/iter; 1.4018x vs baseline; 1.4018x over previous
import jax
import jax.numpy as jnp
from jax import lax
from jax.experimental import pallas as pl
from jax.experimental.pallas import tpu as pltpu

N_DEV = 4
B_LOC = 2
SQ = 128
SKV = 128
HB = 64
HQ = 16
H_LOC = 4
DH = 64
D_MODEL = 512
HD_LOC = H_LOC * DH

BF16 = jnp.bfloat16
F32 = jnp.float32


def kernel(x, Wq, K_ext, V_ext, Wo):
    my = lax.axis_index("i")

    k_loc = lax.dynamic_slice_in_dim(K_ext, B_LOC * my, B_LOC, axis=0)
    v_loc = lax.dynamic_slice_in_dim(V_ext, B_LOC * my, B_LOC, axis=0)
    k_loc = jnp.transpose(k_loc, (0, 2, 1, 3)).reshape(B_LOC * HQ, SKV, DH)
    v_loc = jnp.transpose(v_loc, (0, 2, 1, 3)).reshape(B_LOC * HQ, SKV, DH)
    k_loc = k_loc.astype(BF16)
    v_loc = v_loc.astype(BF16)

    def body(x_ref, wq_ref, k_vmem, v_vmem, wo_ref, out_ref,
             wq_all, wo_all,
             wq_ssem, wq_rsem, wo_ssem, wo_rsem):
        my_pos = lax.axis_index("i")
        left = lax.rem(my_pos + N_DEV - 1, N_DEV)
        right = lax.rem(my_pos + 1, N_DEV)
        jm1 = left
        jp1 = right
        jm2 = lax.rem(my_pos + 2, N_DEV)

        wq_all[pl.ds(my_pos, 1)] = wq_ref[...].astype(BF16)[None]
        wo_all[pl.ds(my_pos, 1)] = wo_ref[...].astype(BF16)[None]

        barrier_sem = pltpu.get_barrier_semaphore()
        for nbr in (left, right):
            pl.semaphore_signal(
                barrier_sem, inc=1,
                device_id=(nbr,), device_id_type=pl.DeviceIdType.MESH,
            )
        pl.semaphore_wait(barrier_sem, 2)

        def copy(buf, slot_idx, ssem, rsem, slot, dev):
            return pltpu.make_async_remote_copy(
                src_ref=buf.at[slot_idx],
                dst_ref=buf.at[slot_idx],
                send_sem=ssem.at[slot],
                recv_sem=rsem.at[slot],
                device_id=(dev,),
                device_id_type=pl.DeviceIdType.MESH,
            )

        q_r0 = copy(wq_all, my_pos, wq_ssem, wq_rsem, 0, right)
        o_l0 = copy(wo_all, my_pos, wo_ssem, wo_rsem, 1, left)
        o_r0 = copy(wo_all, my_pos, wo_ssem, wo_rsem, 0, right)
        q_l0 = copy(wq_all, my_pos, wq_ssem, wq_rsem, 1, left)
        q_r0.start()
        o_l0.start()
        o_r0.start()
        q_l0.start()

        q_recv0 = copy(wq_all, jm1, wq_ssem, wq_rsem, 0, left)
        q_recv1 = copy(wq_all, jp1, wq_ssem, wq_rsem, 1, right)
        q_recv2 = copy(wq_all, jm2, wq_ssem, wq_rsem, 2, left)
        o_recv0 = copy(wo_all, jm1, wo_ssem, wo_rsem, 0, left)
        o_recv1 = copy(wo_all, jp1, wo_ssem, wo_rsem, 1, right)
        o_recv2 = copy(wo_all, jm2, wo_ssem, wo_rsem, 2, right)

        xs = [(x_ref[b] * 0.125).astype(BF16) for b in range(B_LOC)]

        def block_contrib(j, accs):
            wq_j = wq_all[pl.ds(j, 1)].reshape(D_MODEL, HD_LOC)
            wo_j = wo_all[pl.ds(j, 1)].reshape(HD_LOC, D_MODEL)
            out = []
            for b in range(B_LOC):
                q_blk = lax.dot_general(
                    xs[b], wq_j, (((1,), (0,)), ((), ())),
                    preferred_element_type=F32,
                ).astype(BF16)
                ctx_t, ctx_b = [], []
                for r in range(H_LOC):
                    h_idx = b * HQ + j * H_LOC + r
                    k = k_vmem[pl.ds(h_idx, 1)].reshape(SKV, DH)
                    v = v_vmem[pl.ds(h_idx, 1)].reshape(SKV, DH)
                    qt = q_blk[:HB, r * DH:(r + 1) * DH]
                    qb = q_blk[HB:, r * DH:(r + 1) * DH]
                    st = lax.dot_general(
                        qt, k[:HB], (((1,), (1,)), ((), ())),
                        preferred_element_type=F32)
                    sb = lax.dot_general(
                        qb, k, (((1,), (1,)), ((), ())),
                        preferred_element_type=F32)
                    et = jnp.exp(st)
                    eb = jnp.exp(sb)
                    rt = 1.0 / jnp.sum(et, axis=-1, keepdims=True)
                    rb = 1.0 / jnp.sum(eb, axis=-1, keepdims=True)
                    ct = lax.dot_general(
                        et.astype(BF16), v[:HB], (((1,), (0,)), ((), ())),
                        preferred_element_type=F32)
                    cb = lax.dot_general(
                        eb.astype(BF16), v, (((1,), (0,)), ((), ())),
                        preferred_element_type=F32)
                    ctx_t.append((ct * rt).astype(BF16))
                    ctx_b.append((cb * rb).astype(BF16))
                cat_t = jnp.concatenate(ctx_t, axis=1)
                cat_b = jnp.concatenate(ctx_b, axis=1)
                at, ab = accs[b]
                out.append((
                    at + lax.dot_general(
                        cat_t, wo_j, (((1,), (0,)), ((), ())),
                        preferred_element_type=F32),
                    ab + lax.dot_general(
                        cat_b, wo_j, (((1,), (0,)), ((), ())),
                        preferred_element_type=F32),
                ))
            return out

        accs = [(jnp.zeros((HB, D_MODEL), F32),
                 jnp.zeros((HB, D_MODEL), F32)) for _ in range(B_LOC)]

        accs = block_contrib(my_pos, accs)

        q_recv0.wait_recv()
        q_f = copy(wq_all, jm1, wq_ssem, wq_rsem, 2, right)
        q_f.start()
        o_recv1.wait_recv()
        o_f = copy(wo_all, jp1, wo_ssem, wo_rsem, 2, left)
        o_f.start()

        o_recv0.wait_recv()
        accs = block_contrib(jm1, accs)

        q_recv1.wait_recv()
        accs = block_contrib(jp1, accs)

        q_recv2.wait_recv()
        o_recv2.wait_recv()
        accs = block_contrib(jm2, accs)

        for b in range(B_LOC):
            out_ref[b] = jnp.concatenate([accs[b][0], accs[b][1]], axis=0)

        for d in (q_r0, q_l0, o_r0, o_l0, q_f, o_f):
            d.wait_send()

    return pl.pallas_call(
        body,
        out_shape=jax.ShapeDtypeStruct((B_LOC, SQ, D_MODEL), F32),
        in_specs=[
            pl.BlockSpec(memory_space=pltpu.VMEM),
            pl.BlockSpec(memory_space=pltpu.VMEM),
            pl.BlockSpec(memory_space=pltpu.VMEM),
            pl.BlockSpec(memory_space=pltpu.VMEM),
            pl.BlockSpec(memory_space=pltpu.VMEM),
        ],
        out_specs=pl.BlockSpec(memory_space=pltpu.VMEM),
        scratch_shapes=[
            pltpu.VMEM((N_DEV, D_MODEL, HD_LOC), BF16),
            pltpu.VMEM((N_DEV, HD_LOC, D_MODEL), BF16),
            pltpu.SemaphoreType.DMA((3,)),
            pltpu.SemaphoreType.DMA((3,)),
            pltpu.SemaphoreType.DMA((3,)),
            pltpu.SemaphoreType.DMA((3,)),
        ],
        compiler_params=pltpu.CompilerParams(collective_id=0),
    )(x, Wq, k_loc, v_loc, Wo)


# device time: 18576 ns/iter; 1.5936x vs baseline; 1.1368x over previous
import jax
import jax.numpy as jnp
from jax import lax
from jax.experimental import pallas as pl
from jax.experimental.pallas import tpu as pltpu

N_DEV = 4
B_LOC = 2
SQ = 128
SKV = 128
HB = 64
HQ = 16
H_LOC = 4
DH = 64
D_MODEL = 512
HD_LOC = H_LOC * DH

BF16 = jnp.bfloat16
F32 = jnp.float32


def kernel(x, Wq, K_ext, V_ext, Wo):
    my = lax.axis_index("i")

    k_loc = lax.dynamic_slice_in_dim(
        K_ext.reshape(8, SKV, HQ * DH), B_LOC * my, B_LOC, axis=0).astype(BF16)
    v_loc = lax.dynamic_slice_in_dim(
        V_ext.reshape(8, SKV, HQ * DH), B_LOC * my, B_LOC, axis=0).astype(BF16)

    def body(x_ref, wq_ref, k_ref, v_ref, wo_ref, out_ref,
             wq_all, wo_all,
             wq_ssem, wq_rsem, wo_ssem, wo_rsem):
        my_pos = lax.axis_index("i")
        left = lax.rem(my_pos + N_DEV - 1, N_DEV)
        right = lax.rem(my_pos + 1, N_DEV)
        jm1 = left
        jp1 = right
        jm2 = lax.rem(my_pos + 2, N_DEV)

        wq_all[pl.ds(my_pos, 1)] = wq_ref[...].astype(BF16)[None]
        wo_all[pl.ds(my_pos, 1)] = wo_ref[...].astype(BF16)[None]

        barrier_sem = pltpu.get_barrier_semaphore()
        for nbr in (left, right):
            pl.semaphore_signal(
                barrier_sem, inc=1,
                device_id=(nbr,), device_id_type=pl.DeviceIdType.MESH,
            )
        pl.semaphore_wait(barrier_sem, 2)

        def copy(buf, slot_idx, ssem, rsem, slot, dev):
            return pltpu.make_async_remote_copy(
                src_ref=buf.at[slot_idx],
                dst_ref=buf.at[slot_idx],
                send_sem=ssem.at[slot],
                recv_sem=rsem.at[slot],
                device_id=(dev,),
                device_id_type=pl.DeviceIdType.MESH,
            )

        q_r0 = copy(wq_all, my_pos, wq_ssem, wq_rsem, 0, right)
        o_l0 = copy(wo_all, my_pos, wo_ssem, wo_rsem, 1, left)
        o_r0 = copy(wo_all, my_pos, wo_ssem, wo_rsem, 0, right)
        q_l0 = copy(wq_all, my_pos, wq_ssem, wq_rsem, 1, left)
        q_r0.start()
        o_l0.start()
        o_r0.start()
        q_l0.start()

        q_recv0 = copy(wq_all, jm1, wq_ssem, wq_rsem, 0, left)
        q_recv1 = copy(wq_all, jp1, wq_ssem, wq_rsem, 1, right)
        q_recv2 = copy(wq_all, jm2, wq_ssem, wq_rsem, 2, left)
        o_recv0 = copy(wo_all, jm1, wo_ssem, wo_rsem, 0, left)
        o_recv1 = copy(wo_all, jp1, wo_ssem, wo_rsem, 1, right)
        o_recv2 = copy(wo_all, jm2, wo_ssem, wo_rsem, 2, right)

        xs = [(x_ref[b] * 0.125).astype(BF16) for b in range(B_LOC)]

        def ctx_phase(j):
            wq_j = wq_all[pl.ds(j, 1)].reshape(D_MODEL, HD_LOC)
            res = []
            for b in range(B_LOC):
                q_blk = lax.dot_general(
                    xs[b], wq_j, (((1,), (0,)), ((), ())),
                    preferred_element_type=F32,
                ).astype(BF16)
                kbj = k_ref[b, :, pl.ds(j * HD_LOC, HD_LOC)]
                vbj = v_ref[b, :, pl.ds(j * HD_LOC, HD_LOC)]
                ctx_t, ctx_b = [], []
                for r in range(H_LOC):
                    sl = slice(r * DH, (r + 1) * DH)
                    k = kbj[:, sl]
                    v = vbj[:, sl]
                    qt = q_blk[:HB, sl]
                    qb = q_blk[HB:, sl]
                    st = lax.dot_general(
                        qt, k[:HB], (((1,), (1,)), ((), ())),
                        preferred_element_type=F32)
                    sb = lax.dot_general(
                        qb, k, (((1,), (1,)), ((), ())),
                        preferred_element_type=F32)
                    et = jnp.exp(st)
                    eb = jnp.exp(sb)
                    rt = 1.0 / jnp.sum(et, axis=-1, keepdims=True)
                    rb = 1.0 / jnp.sum(eb, axis=-1, keepdims=True)
                    ct = lax.dot_general(
                        et.astype(BF16), v[:HB], (((1,), (0,)), ((), ())),
                        preferred_element_type=F32)
                    cb = lax.dot_general(
                        eb.astype(BF16), v, (((1,), (0,)), ((), ())),
                        preferred_element_type=F32)
                    ctx_t.append((ct * rt).astype(BF16))
                    ctx_b.append((cb * rb).astype(BF16))
                res.append((jnp.concatenate(ctx_t, axis=1),
                            jnp.concatenate(ctx_b, axis=1)))
            return res

        def proj(j, ctxs, accs):
            wo_j = wo_all[pl.ds(j, 1)].reshape(HD_LOC, D_MODEL)
            out = []
            for b in range(B_LOC):
                (cat_t, cat_b), (at, ab) = ctxs[b], accs[b]
                out.append((
                    at + lax.dot_general(
                        cat_t, wo_j, (((1,), (0,)), ((), ())),
                        preferred_element_type=F32),
                    ab + lax.dot_general(
                        cat_b, wo_j, (((1,), (0,)), ((), ())),
                        preferred_element_type=F32),
                ))
            return out

        accs = [(jnp.zeros((HB, D_MODEL), F32),
                 jnp.zeros((HB, D_MODEL), F32)) for _ in range(B_LOC)]

        accs = proj(my_pos, ctx_phase(my_pos), accs)

        q_recv0.wait_recv()
        q_f = copy(wq_all, jm1, wq_ssem, wq_rsem, 2, right)
        q_f.start()
        o_recv1.wait_recv()
        o_f = copy(wo_all, jp1, wo_ssem, wo_rsem, 2, left)
        o_f.start()

        c_jm1 = ctx_phase(jm1)
        o_recv0.wait_recv()
        accs = proj(jm1, c_jm1, accs)

        q_recv1.wait_recv()
        accs = proj(jp1, ctx_phase(jp1), accs)

        q_recv2.wait_recv()
        c_jm2 = ctx_phase(jm2)
        o_recv2.wait_recv()
        accs = proj(jm2, c_jm2, accs)

        for b in range(B_LOC):
            out_ref[b] = jnp.concatenate([accs[b][0], accs[b][1]], axis=0)

        for d in (q_r0, q_l0, o_r0, o_l0, q_f, o_f):
            d.wait_send()

    return pl.pallas_call(
        body,
        out_shape=jax.ShapeDtypeStruct((B_LOC, SQ, D_MODEL), F32),
        in_specs=[
            pl.BlockSpec(memory_space=pltpu.VMEM),
            pl.BlockSpec(memory_space=pltpu.VMEM),
            pl.BlockSpec(memory_space=pltpu.VMEM),
            pl.BlockSpec(memory_space=pltpu.VMEM),
            pl.BlockSpec(memory_space=pltpu.VMEM),
        ],
        out_specs=pl.BlockSpec(memory_space=pltpu.VMEM),
        scratch_shapes=[
            pltpu.VMEM((N_DEV, D_MODEL, HD_LOC), BF16),
            pltpu.VMEM((N_DEV, HD_LOC, D_MODEL), BF16),
            pltpu.SemaphoreType.DMA((3,)),
            pltpu.SemaphoreType.DMA((3,)),
            pltpu.SemaphoreType.DMA((3,)),
            pltpu.SemaphoreType.DMA((3,)),
        ],
        compiler_params=pltpu.CompilerParams(collective_id=0),
    )(x, Wq, k_loc, v_loc, Wo)
